# baseline (speedup 1.0000x reference)
import functools

import jax
import jax.numpy as jnp
from jax import lax
from jax.experimental import pallas as pl
from jax.experimental.pallas import tpu as pltpu
from jax.experimental.pallas import tpu_sc as plsc

_C = 100000
_D = 128
_B = 16384
_NC = 2
_NS = 16
_L = 16
_NK = _D // _L
_NW = _NC * _NS
_BW = _B // _NW
_CH = 128
_NCHUNK = _BW // _CH
_NG = _CH // _L


def _sc_body(table, left, right, sl, sr, out,
             idxl_v, idxr_v, slv, srv,
             lrows0, rrows0, lrows1, rrows1, outv, sem0, sem1):
    wid = lax.axis_index("s") * _NC + lax.axis_index("c")
    base = wid * _BW

    pltpu.sync_copy(left.at[pl.ds(base, _BW)], idxl_v)
    pltpu.sync_copy(right.at[pl.ds(base, _BW)], idxr_v)

    lbufs = (lrows0, lrows1)
    rbufs = (rrows0, rrows1)
    sems = (sem0, sem1)

    def issue(c, par):
        coff = pl.multiple_of(c * _CH, _CH)
        cl = pltpu.async_copy(
            table.at[idxl_v.at[pl.ds(coff, _CH)]], lbufs[par], sems[par])
        cr = pltpu.async_copy(
            table.at[idxr_v.at[pl.ds(coff, _CH)]], rbufs[par], sems[par])
        return cl, cr

    issue(0, 0)
    issue(1, 1)

    pltpu.sync_copy(sl.at[pl.ds(base, _BW)], slv)
    pltpu.sync_copy(sr.at[pl.ds(base, _BW)], srv)

    def det_body(g, m):
        gb = pl.multiple_of(g * _L, _L)
        m = jnp.maximum(m, idxl_v[pl.ds(gb, _L)])
        return jnp.maximum(m, idxr_v[pl.ds(gb, _L)])

    mx = lax.fori_loop(0, _BW // _L, det_body,
                       jnp.zeros((_L,), jnp.int32))
    mxs = mx[0]
    for lane in range(1, _L):
        mxs = jnp.maximum(mxs, mx[lane])

    def pair_body(i, accs):
        c0 = i * 2
        for par in range(2):
            c = c0 + par
            coff = pl.multiple_of(c * _CH, _CH)
            lrows_v = lbufs[par]
            rrows_v = rbufs[par]
            pltpu.make_async_copy(
                table.at[idxl_v.at[pl.ds(coff, _CH)]], lrows_v, sems[par]
            ).wait()
            pltpu.make_async_copy(
                table.at[idxr_v.at[pl.ds(coff, _CH)]], rrows_v, sems[par]
            ).wait()

            @pl.when(mxs >= _C - 2)
            def _patch():
                def patch_row(buf, row, idx):
                    @pl.when(idx >= _C - 2)
                    def _():
                        is_last = idx == _C - 1

                        def kb(k, carry):
                            off = pl.multiple_of(k * _L, _L)
                            x = buf[row, pl.ds(off, _L)]
                            p1 = jnp.where(x > 0.0, 1.0, x)
                            p2 = jnp.where(x < 1.0, 0.0, x)
                            buf[row, pl.ds(off, _L)] = jnp.where(
                                is_last, p1, p2)
                            return carry

                        lax.fori_loop(0, _NK, kb, 0)

                def gbody(g, carry):
                    gb = pl.multiple_of(g * _L, _L)
                    ilv = idxl_v[pl.ds(coff + gb, _L)]
                    irv = idxr_v[pl.ds(coff + gb, _L)]
                    for lane in range(_L):
                        patch_row(lrows_v, gb + lane, ilv[lane])
                        patch_row(rrows_v, gb + lane, irv[lane])
                    return carry

                lax.fori_loop(0, _NG, gbody, 0)

            def group_body(g, accs, lrows_v=lrows_v, rrows_v=rrows_v,
                           coff=coff):
                gb = pl.multiple_of(g * _L, _L)
                slv16 = slv[pl.ds(coff + gb, _L)]
                srv16 = srv[pl.ds(coff + gb, _L)]
                for lane in range(_L):
                    slb = slv16[lane]
                    srb = srv16[lane]
                    row = gb + lane
                    lv = tuple(
                        lrows_v[row, pl.ds(k * _L, _L)] for k in range(_NK))
                    rv = tuple(
                        rrows_v[row, pl.ds(k * _L, _L)] for k in range(_NK))
                    accs = tuple(
                        a + jnp.maximum(slb * l - srb * r, 0.0)
                        for a, l, r in zip(accs, lv, rv)
                    )
                return accs

            accs = lax.fori_loop(0, _NG, group_body, accs)

            @pl.when(c + 2 < _NCHUNK)
            def _prefetch():
                issue(c + 2, par)

        return accs

    accs = lax.fori_loop(
        0, _NCHUNK // 2, pair_body,
        tuple(jnp.zeros((_L,), jnp.float32) for _ in range(_NK)),
    )

    total = accs[0]
    for k in range(1, _NK):
        total = total + accs[k]
    outv[...] = total
    pltpu.sync_copy(outv, out.at[pl.ds(wid * _L, _L)])


_sc_call = functools.partial(
    pl.kernel,
    out_type=jax.ShapeDtypeStruct((_NW * _L,), jnp.float32),
    mesh=plsc.VectorSubcoreMesh(
        core_axis_name="c", subcore_axis_name="s",
        num_cores=_NC, num_subcores=_NS,
    ),
    scratch_types=[
        pltpu.VMEM((_BW,), jnp.int32),
        pltpu.VMEM((_BW,), jnp.int32),
        pltpu.VMEM((_BW,), jnp.float32),
        pltpu.VMEM((_BW,), jnp.float32),
        pltpu.VMEM((_CH, _D), jnp.float32),
        pltpu.VMEM((_CH, _D), jnp.float32),
        pltpu.VMEM((_CH, _D), jnp.float32),
        pltpu.VMEM((_CH, _D), jnp.float32),
        pltpu.VMEM((_L,), jnp.float32),
        pltpu.SemaphoreType.DMA,
        pltpu.SemaphoreType.DMA,
    ],
)(_sc_body)


def kernel(cEmb, left, right, negf, atype):
    del atype
    left = left.astype(jnp.int32)
    right = right.astype(jnp.int32)
    sl = (2 * negf[:, 0] + 1).astype(jnp.float32)
    sr = (1 - 2 * negf[:, 1]).astype(jnp.float32)
    partials = _sc_call(cEmb, left, right, sl, sr)
    return jnp.sum(partials) * (1.0 / _B)


# speedup vs baseline: 1.8751x; 1.8751x over previous
import functools

import jax
import jax.numpy as jnp
from jax import lax
from jax.experimental import pallas as pl
from jax.experimental.pallas import tpu as pltpu
from jax.experimental.pallas import tpu_sc as plsc

_C = 100000
_D = 128
_B = 16384
_NC = 2
_NS = 16
_L = 16
_NK = _D // _L
_NW = _NC * _NS
_BW = _B // _NW
_CH = 128
_NCHUNK = _BW // _CH
_NG = _CH // _L


def _sc_body(table, left, right, sl, sr, out,
             idxl_v, idxr_v, slv, srv,
             lrows0, rrows0, lrows1, rrows1, outv, sem0, sem1):
    wid = lax.axis_index("s") * _NC + lax.axis_index("c")
    base = wid * _BW

    pltpu.sync_copy(left.at[pl.ds(base, _BW)], idxl_v)
    pltpu.sync_copy(right.at[pl.ds(base, _BW)], idxr_v)

    lbufs = (lrows0, lrows1)
    rbufs = (rrows0, rrows1)
    sems = (sem0, sem1)

    def issue(c, par):
        coff = pl.multiple_of(c * _CH, _CH)
        cl = pltpu.async_copy(
            table.at[idxl_v.at[pl.ds(coff, _CH)]], lbufs[par], sems[par])
        cr = pltpu.async_copy(
            table.at[idxr_v.at[pl.ds(coff, _CH)]], rbufs[par], sems[par])
        return cl, cr

    issue(0, 0)
    issue(1, 1)

    pltpu.sync_copy(sl.at[pl.ds(base, _BW)], slv)
    pltpu.sync_copy(sr.at[pl.ds(base, _BW)], srv)

    def det_body(g, m):
        gb = pl.multiple_of(g * _L, _L)
        m = jnp.maximum(m, idxl_v[pl.ds(gb, _L)])
        return jnp.maximum(m, idxr_v[pl.ds(gb, _L)])

    mx = lax.fori_loop(0, _BW // _L, det_body,
                       jnp.zeros((_L,), jnp.int32))
    mxs = mx[0]
    for lane in range(1, _L):
        mxs = jnp.maximum(mxs, mx[lane])

    def pair_body(i, accs):
        c0 = i * 2
        for par in range(2):
            c = c0 + par
            coff = pl.multiple_of(c * _CH, _CH)
            lrows_v = lbufs[par]
            rrows_v = rbufs[par]
            pltpu.make_async_copy(
                table.at[idxl_v.at[pl.ds(coff, _CH)]], lrows_v, sems[par]
            ).wait()
            pltpu.make_async_copy(
                table.at[idxr_v.at[pl.ds(coff, _CH)]], rrows_v, sems[par]
            ).wait()

            @pl.when(mxs >= _C - 2)
            def _patch():
                def patch_row(buf, row, idx):
                    @pl.when(idx >= _C - 2)
                    def _():
                        is_last = idx == _C - 1

                        def kb(k, carry):
                            off = pl.multiple_of(k * _L, _L)
                            x = buf[row, pl.ds(off, _L)]
                            p1 = jnp.where(x > 0.0, 1.0, x)
                            p2 = jnp.where(x < 1.0, 0.0, x)
                            buf[row, pl.ds(off, _L)] = jnp.where(
                                is_last, p1, p2)
                            return carry

                        lax.fori_loop(0, _NK, kb, 0)

                def gbody(g, carry):
                    gb = pl.multiple_of(g * _L, _L)
                    ilv = idxl_v[pl.ds(coff + gb, _L)]
                    irv = idxr_v[pl.ds(coff + gb, _L)]
                    for lane in range(_L):
                        patch_row(lrows_v, gb + lane, ilv[lane])
                        patch_row(rrows_v, gb + lane, irv[lane])
                    return carry

                lax.fori_loop(0, _NG, gbody, 0)

            def group_body(g, accs, lrows_v=lrows_v, rrows_v=rrows_v,
                           coff=coff):
                gb = pl.multiple_of(g * _L, _L)
                slv16 = slv[pl.ds(coff + gb, _L)]
                srv16 = srv[pl.ds(coff + gb, _L)]
                for lane in range(_L):
                    slb = slv16[lane]
                    srb = srv16[lane]
                    row = gb + lane
                    lv = tuple(
                        lrows_v[row, pl.ds(k * _L, _L)] for k in range(_NK))
                    rv = tuple(
                        rrows_v[row, pl.ds(k * _L, _L)] for k in range(_NK))
                    accs = tuple(
                        a + jnp.maximum(slb * l - srb * r, 0.0)
                        for a, l, r in zip(accs, lv, rv)
                    )
                return accs

            accs = lax.fori_loop(0, _NG, group_body, accs, unroll=2)

            @pl.when(c + 2 < _NCHUNK)
            def _prefetch():
                issue(c + 2, par)

        return accs

    accs = lax.fori_loop(
        0, _NCHUNK // 2, pair_body,
        tuple(jnp.zeros((_L,), jnp.float32) for _ in range(_NK)),
    )

    total = accs[0]
    for k in range(1, _NK):
        total = total + accs[k]
    outv[...] = total
    pltpu.sync_copy(outv, out.at[pl.ds(wid * _L, _L)])


_sc_call = functools.partial(
    pl.kernel,
    out_type=jax.ShapeDtypeStruct((_NW * _L,), jnp.float32),
    mesh=plsc.VectorSubcoreMesh(
        core_axis_name="c", subcore_axis_name="s",
        num_cores=_NC, num_subcores=_NS,
    ),
    scratch_types=[
        pltpu.VMEM((_BW,), jnp.int32),
        pltpu.VMEM((_BW,), jnp.int32),
        pltpu.VMEM((_BW,), jnp.float32),
        pltpu.VMEM((_BW,), jnp.float32),
        pltpu.VMEM((_CH, _D), jnp.float32),
        pltpu.VMEM((_CH, _D), jnp.float32),
        pltpu.VMEM((_CH, _D), jnp.float32),
        pltpu.VMEM((_CH, _D), jnp.float32),
        pltpu.VMEM((_L,), jnp.float32),
        pltpu.SemaphoreType.DMA,
        pltpu.SemaphoreType.DMA,
    ],
)(_sc_body)


def kernel(cEmb, left, right, negf, atype):
    del atype
    left = left.astype(jnp.int32)
    right = right.astype(jnp.int32)
    sl = (2 * negf[:, 0] + 1).astype(jnp.float32)
    sr = (1 - 2 * negf[:, 1]).astype(jnp.float32)
    partials = _sc_call(cEmb, left, right, sl, sr)
    return jnp.sum(partials) * (1.0 / _B)
